# Initial kernel scaffold; baseline (speedup 1.0000x reference)
import jax
import jax.numpy as jnp
from jax.experimental import pallas as pl


def kernel(x, edge_index, W_in, Wl_in, b_in, blocks_W, blocks_Wl, blocks_b, W_out, Wl_out, b_out):
    raise NotImplementedError("write your pallas kernel here")


# speedup vs baseline: 7.1607x; 7.1607x over previous
import functools

import jax
import jax.numpy as jnp
from jax import lax
from jax.experimental import pallas as pl
from jax.experimental.pallas import tpu as pltpu
from jax.experimental.pallas import tpu_sc as plsc

N = 10000
E = 320000
D = 128
BLOCKS = 6

NC = 2
NS = 16
NW = NC * NS
NPAD = 10240
ROWS_PER_SUB = NPAD // NS
EDGES_PER_W = E // NW
CHUNK = 80
NCHUNKS = EDGES_PER_W // CHUNK


def _segsum_body(z_hbm, src_hbm, dst_hbm, zeros_hbm, out0_hbm, out1_hbm,
                 idx_s0, idx_d0, idx_s1, idx_d1, rows0, rows1, zbuf, acc,
                 sem0, sem1):
    c = lax.axis_index("c")
    s = lax.axis_index("s")
    w = s * NC + c
    base = w * EDGES_PER_W

    pltpu.sync_copy(zeros_hbm, zbuf)
    for j in range(ROWS_PER_SUB // 128):
        pltpu.sync_copy(zbuf, acc.at[pl.ds(s * ROWS_PER_SUB + j * 128, 128)])
    plsc.subcore_barrier()

    def load_idx(i, idx_s, idx_d):
        off = base + i * CHUNK
        pltpu.sync_copy(src_hbm.at[pl.ds(off, CHUNK)], idx_s)
        pltpu.sync_copy(dst_hbm.at[pl.ds(off, CHUNK)], idx_d)

    def gather_start(idx_s, rows, sem):
        pltpu.async_copy(z_hbm.at[idx_s], rows, sem)

    def gather_wait(idx_s, rows, sem):
        pltpu.make_async_copy(z_hbm.at[idx_s], rows, sem).wait()

    def scatter_add(idx_d, rows):
        pltpu.sync_copy(rows, acc.at[idx_d], add=True)

    load_idx(0, idx_s0, idx_d0)
    gather_start(idx_s0, rows0, sem0)

    def body(k, carry):
        a = 2 * k + 1
        b = 2 * k + 2
        load_idx(a, idx_s1, idx_d1)
        gather_start(idx_s1, rows1, sem1)
        gather_wait(idx_s0, rows0, sem0)
        scatter_add(idx_d0, rows0)
        load_idx(b, idx_s0, idx_d0)
        gather_start(idx_s0, rows0, sem0)
        gather_wait(idx_s1, rows1, sem1)
        scatter_add(idx_d1, rows1)
        return carry

    lax.fori_loop(0, (NCHUNKS - 1) // 2, body, 0)
    gather_wait(idx_s0, rows0, sem0)
    scatter_add(idx_d0, rows0)

    plsc.subcore_barrier()

    sl = pl.ds(s * ROWS_PER_SUB, ROWS_PER_SUB)

    @pl.when(c == 0)
    def _():
        pltpu.sync_copy(acc.at[sl], out0_hbm.at[sl])

    @pl.when(c == 1)
    def _():
        pltpu.sync_copy(acc.at[sl], out1_hbm.at[sl])


@functools.partial(
    pl.kernel,
    mesh=plsc.VectorSubcoreMesh(core_axis_name="c", subcore_axis_name="s"),
    out_type=[
        jax.ShapeDtypeStruct((NPAD, D), jnp.float32),
        jax.ShapeDtypeStruct((NPAD, D), jnp.float32),
    ],
    scratch_types=[
        pltpu.VMEM((CHUNK,), jnp.int32),
        pltpu.VMEM((CHUNK,), jnp.int32),
        pltpu.VMEM((CHUNK,), jnp.int32),
        pltpu.VMEM((CHUNK,), jnp.int32),
        pltpu.VMEM((CHUNK, D), jnp.float32),
        pltpu.VMEM((CHUNK, D), jnp.float32),
        pltpu.VMEM((128, D), jnp.float32),
        pltpu.VMEM_SHARED((NPAD, D), jnp.float32),
        pltpu.SemaphoreType.DMA,
        pltpu.SemaphoreType.DMA,
    ],
)
def _segsum(z_hbm, src_hbm, dst_hbm, zeros_hbm, out0_hbm, out1_hbm, *scratch):
    _segsum_body(z_hbm, src_hbm, dst_hbm, zeros_hbm, out0_hbm, out1_hbm,
                 *scratch)


ROWS_BLK = 1000


def _combine_kernel(u0_ref, u1_ref, z_ref, w_ref, wl_ref, b_ref, out_ref):
    acc = jnp.dot(u0_ref[...] + u1_ref[...], w_ref[...],
                  preferred_element_type=jnp.float32)
    acc = acc + jnp.dot(z_ref[...], wl_ref[...],
                        preferred_element_type=jnp.float32)
    out_ref[...] = acc + b_ref[...]


def _combine_res_kernel(u0_ref, u1_ref, z_ref, w_ref, wl_ref, b_ref, h_ref,
                        out_ref):
    acc = jnp.dot(u0_ref[...] + u1_ref[...], w_ref[...],
                  preferred_element_type=jnp.float32)
    acc = acc + jnp.dot(z_ref[...], wl_ref[...],
                        preferred_element_type=jnp.float32)
    out_ref[...] = (h_ref[...] + acc + b_ref[...]) * 0.5


_row_spec = pl.BlockSpec((ROWS_BLK, D), lambda i: (i, 0))
_mat_spec = pl.BlockSpec((D, D), lambda i: (0, 0))
_bias_spec = pl.BlockSpec((1, D), lambda i: (0, 0))


def _combine(u0, u1, z, W, Wl, b):
    return pl.pallas_call(
        _combine_kernel,
        grid=(N // ROWS_BLK,),
        in_specs=[_row_spec, _row_spec, _row_spec, _mat_spec, _mat_spec,
                  _bias_spec],
        out_specs=_row_spec,
        out_shape=jax.ShapeDtypeStruct((N, D), jnp.float32),
    )(u0, u1, z, W, Wl, b.reshape(1, D))


def _combine_res(u0, u1, z, W, Wl, b, h):
    return pl.pallas_call(
        _combine_res_kernel,
        grid=(N // ROWS_BLK,),
        in_specs=[_row_spec, _row_spec, _row_spec, _mat_spec, _mat_spec,
                  _bias_spec, _row_spec],
        out_specs=_row_spec,
        out_shape=jax.ShapeDtypeStruct((N, D), jnp.float32),
    )(u0, u1, z, W, Wl, b.reshape(1, D), h)


def kernel(x, edge_index, W_in, Wl_in, b_in, blocks_W, blocks_Wl, blocks_b,
           W_out, Wl_out, b_out):
    src = edge_index[0]
    dst = edge_index[1]
    zeros = jnp.zeros((128, D), jnp.float32)

    def gconv(z, W, Wl, b):
        u0, u1 = _segsum(z, src, dst, zeros)
        return _combine(u0, u1, z, W, Wl, b)

    def gconv_res(z, W, Wl, b, h):
        u0, u1 = _segsum(z, src, dst, zeros)
        return _combine_res(u0, u1, z, W, Wl, b, h)

    h = gconv(x, W_in, Wl_in, b_in)
    for i in range(BLOCKS):
        t = gconv(h, blocks_W[i, 0], blocks_Wl[i, 0], blocks_b[i, 0])
        h = gconv_res(t, blocks_W[i, 1], blocks_Wl[i, 1], blocks_b[i, 1], h)
    x_out = gconv(h, W_out, Wl_out, b_out)
    return (x_out, h)
